# baseline (speedup 1.0000x reference)
import functools

import jax
import jax.numpy as jnp
from jax import lax
from jax.experimental import pallas as pl
from jax.experimental.pallas import tpu as pltpu
from jax.experimental.pallas import tpu_sc as plsc

NUM_ITEMS = 100000
NUM_META = 1000
DM = 64
MT = 4
B = 16384
KNEG = 5
SLOTS = KNEG + 1

NC, NS, L = 2, 16, 16
NW = NC * NS
BPW = B // NW
SPT = BPW * SLOTS
CH = 128
NCHUNK = SPT // CH
GPC = CH // L

_mesh = plsc.VectorSubcoreMesh(core_axis_name="c", subcore_axis_name="s")


@functools.partial(
    pl.kernel,
    out_type=jax.ShapeDtypeStruct((B * SLOTS,), jnp.float32),
    mesh=_mesh,
    scratch_types=[
        pltpu.VMEM((BPW * DM,), jnp.float32),
        pltpu.VMEM((NUM_META * DM,), jnp.float32),
        pltpu.VMEM((SPT,), jnp.int32),
        pltpu.VMEM((SPT,), jnp.int32),
        pltpu.VMEM((CH * MT,), jnp.int32),
        pltpu.VMEM((CH, DM), jnp.float32),
        pltpu.VMEM((CH * MT,), jnp.int32),
        pltpu.VMEM((CH * MT,), jnp.float32),
        pltpu.VMEM((CH * MT,), jnp.int32),
        pltpu.VMEM((CH, DM), jnp.float32),
        pltpu.VMEM((CH * MT,), jnp.int32),
        pltpu.VMEM((CH * MT,), jnp.float32),
        pltpu.VMEM((SPT,), jnp.float32),
    ] + [pltpu.SemaphoreType.DMA] * 18,
    compiler_params=pltpu.CompilerParams(
        use_tc_tiling_on_sc=False, needs_layout_passes=False),
)
def _score_kernel(h_hbm, idx_hbm, idx4_hbm, item_hbm, meta_hbm, mi_hbm,
                  mw_hbm, hrow_hbm, out_hbm,
                  h_v, meta_v, idx_v, hrow_v,
                  idx4_a, rows_a, mi4_a, mw4_a,
                  idx4_b, rows_b, mi4_b, mw4_b,
                  out_v, *sems):
    wid = lax.axis_index("s") * NC + lax.axis_index("c")
    row0 = wid * BPW
    slot0 = row0 * SLOTS

    pltpu.sync_copy(h_hbm.at[pl.ds(row0 * DM, BPW * DM)], h_v)
    pltpu.sync_copy(meta_hbm, meta_v)
    pltpu.sync_copy(idx_hbm.at[pl.ds(slot0, SPT)], idx_v)
    pltpu.sync_copy(hrow_hbm, hrow_v)

    lanes = lax.iota(jnp.int32, L)

    NSPL = 4
    QW = CH * MT // NSPL
    bufs = ((idx4_a, rows_a, mi4_a, mw4_a, sems[0:9]),
            (idx4_b, rows_b, mi4_b, mw4_b, sems[9:18]))

    def issue(c, buf):
        idx4_v, rows_v, mi4_v, mw4_v, s = buf
        pltpu.sync_copy(
            idx4_hbm.at[pl.ds((slot0 + c * CH) * MT, CH * MT)], idx4_v)
        pltpu.async_copy(item_hbm.at[idx_v.at[pl.ds(c * CH, CH)]], rows_v, s[0])
        for q in range(NSPL):
            iq = idx4_v.at[pl.ds(q * QW, QW)]
            pltpu.async_copy(mi_hbm.at[iq], mi4_v.at[pl.ds(q * QW, QW)],
                             s[1 + q])
            pltpu.async_copy(mw_hbm.at[iq], mw4_v.at[pl.ds(q * QW, QW)],
                             s[1 + NSPL + q])

    def drain(buf):
        idx4_v, rows_v, mi4_v, mw4_v, s = buf
        pltpu.make_async_copy(
            item_hbm.at[pl.ds(0, CH)], rows_v, s[0]).wait()
        for q in range(NSPL):
            pltpu.make_async_copy(
                mi_hbm.at[pl.ds(0, QW)],
                mi4_v.at[pl.ds(q * QW, QW)], s[1 + q]).wait()
            pltpu.make_async_copy(
                mw_hbm.at[pl.ds(0, QW)],
                mw4_v.at[pl.ds(q * QW, QW)], s[1 + NSPL + q]).wait()

    def compute(c, buf):
        idx4_v, rows_v, mi4_v, mw4_v, s = buf

        @pl.loop(0, GPC)
        def _group(g):
            sl = g * L + lanes
            off = c * CH + g * L
            hbase = hrow_v[pl.ds(off, L)]
            sl4 = sl * MT
            mbases = []
            ws = []
            for m in range(MT):
                mi_m = plsc.load_gather(mi4_v, [sl4 + m])
                mbases.append(mi_m * DM)
                ws.append(plsc.load_gather(mw4_v, [sl4 + m]))

            accs = [jnp.zeros((L,), jnp.float32) for _ in range(4)]
            for d in range(DM):
                dl = (lanes + d) & (DM - 1)
                hv = plsc.load_gather(h_v, [hbase + dl])
                ev = plsc.load_gather(rows_v, [sl, dl])
                for m in range(MT):
                    ev = ev + ws[m] * plsc.load_gather(meta_v, [mbases[m] + dl])
                accs[d % 4] = accs[d % 4] + hv * ev
            acc = (accs[0] + accs[1]) + (accs[2] + accs[3])
            score = acc * (1.0 / (MT + 1))
            out_v[pl.ds(off, L)] = 1.0 / (1.0 + jnp.exp(-score))

    issue(0, bufs[0])

    @pl.loop(0, NCHUNK, step=2)
    def _chunk(c):
        issue(c + 1, bufs[1])
        drain(bufs[0])
        compute(c, bufs[0])

        @pl.when(c + 2 < NCHUNK)
        def _():
            issue(c + 2, bufs[0])
        drain(bufs[1])
        compute(c + 1, bufs[1])

    pltpu.sync_copy(out_v, out_hbm.at[pl.ds(slot0, SPT)])


def kernel(h, target_index, negative_sample, item_emb, meta_emb,
           item_meta_indicies, item_meta_weights):
    idx_all = jnp.concatenate(
        [target_index[:, None], negative_sample], axis=1
    ).astype(jnp.int32).reshape(-1)
    idx4_all = (idx_all[:, None] * MT
                + jnp.arange(MT, dtype=jnp.int32)[None, :]).reshape(-1)
    hrow_map = ((jnp.arange(SPT, dtype=jnp.int32) // SLOTS) * DM).astype(jnp.int32)

    scores = _score_kernel(
        h.reshape(-1), idx_all, idx4_all, item_emb, meta_emb.reshape(-1),
        item_meta_indicies.astype(jnp.int32).reshape(-1),
        item_meta_weights.reshape(-1), hrow_map,
    ).reshape(B, SLOTS)

    pos_out = scores[:, :1]
    neg_out = scores[:, 1:]
    pos_label = jnp.ones((B, 1), dtype=jnp.float32)
    neg_label = jnp.zeros((B, KNEG), dtype=jnp.float32)
    return pos_out, pos_label, neg_out, neg_label


# speedup vs baseline: 1.1868x; 1.1868x over previous
import functools

import jax
import jax.numpy as jnp
from jax import lax
from jax.experimental import pallas as pl
from jax.experimental.pallas import tpu as pltpu
from jax.experimental.pallas import tpu_sc as plsc

NUM_ITEMS = 100000
NUM_META = 1000
DM = 64
MT = 4
B = 16384
KNEG = 5
SLOTS = KNEG + 1

NC, NS, L = 2, 16, 16
NW = NC * NS
BPW = B // NW
SPT = BPW * SLOTS
CH = 64
NCHUNK = SPT // CH
GPC = CH // L

_mesh = plsc.VectorSubcoreMesh(core_axis_name="c", subcore_axis_name="s")


@functools.partial(
    pl.kernel,
    out_type=jax.ShapeDtypeStruct((B * SLOTS,), jnp.float32),
    mesh=_mesh,
    scratch_types=[
        pltpu.VMEM((BPW * DM,), jnp.float32),
        pltpu.VMEM((NUM_META * DM,), jnp.float32),
        pltpu.VMEM((SPT,), jnp.int32),
        pltpu.VMEM((SPT,), jnp.int32),
        pltpu.VMEM((CH, DM), jnp.float32),
        pltpu.VMEM((CH, DM), jnp.float32),
        pltpu.VMEM((CH, DM), jnp.float32),
        pltpu.VMEM((CH, DM), jnp.float32),
        pltpu.VMEM((SPT,), jnp.float32),
    ] + [pltpu.SemaphoreType.DMA] * 4,
    compiler_params=pltpu.CompilerParams(
        use_tc_tiling_on_sc=False, needs_layout_passes=False),
)
def _score_kernel(h_hbm, idx_hbm, item_hbm, meta_hbm, miw_hbm,
                  hrow_hbm, out_hbm,
                  h_v, meta_v, idx_v, hrow_v,
                  rows_a, miw_a, rows_b, miw_b,
                  out_v, *sems):
    wid = lax.axis_index("s") * NC + lax.axis_index("c")
    row0 = wid * BPW
    slot0 = row0 * SLOTS

    pltpu.sync_copy(h_hbm.at[pl.ds(row0 * DM, BPW * DM)], h_v)
    pltpu.sync_copy(meta_hbm, meta_v)
    pltpu.sync_copy(idx_hbm.at[pl.ds(slot0, SPT)], idx_v)
    pltpu.sync_copy(hrow_hbm, hrow_v)

    lanes = lax.iota(jnp.int32, L)

    bufs = ((rows_a, miw_a, sems[0:2]),
            (rows_b, miw_b, sems[2:4]))

    def issue(c, buf):
        rows_v, miw_v, s = buf
        idx_c = idx_v.at[pl.ds(c * CH, CH)]
        pltpu.async_copy(item_hbm.at[idx_c], rows_v, s[0])
        pltpu.async_copy(miw_hbm.at[idx_c], miw_v, s[1])

    def drain(buf):
        rows_v, miw_v, s = buf
        pltpu.make_async_copy(
            item_hbm.at[pl.ds(0, CH)], rows_v, s[0]).wait()
        pltpu.make_async_copy(
            miw_hbm.at[pl.ds(0, CH)], miw_v, s[1]).wait()

    def compute(c, buf):
        rows_v, miw_v, s = buf

        @pl.loop(0, GPC)
        def _group(g):
            sl = g * L + lanes
            off = c * CH + g * L
            hbase = hrow_v[pl.ds(off, L)]
            mbases = []
            ws = []
            for m in range(MT):
                msp = jnp.full((L,), m, jnp.int32)
                mi_m = plsc.load_gather(miw_v, [sl, msp]).astype(jnp.int32)
                mi_m = jnp.minimum(jnp.maximum(mi_m, 0), NUM_META - 1)
                mbases.append(mi_m * DM)
                ws.append(plsc.load_gather(miw_v, [sl, msp + MT]))

            accs = [jnp.zeros((L,), jnp.float32) for _ in range(4)]
            for d in range(DM):
                dl = (lanes + d) & (DM - 1)
                hv = plsc.load_gather(h_v, [hbase + dl])
                ev = plsc.load_gather(rows_v, [sl, dl])
                for m in range(MT):
                    ev = ev + ws[m] * plsc.load_gather(meta_v, [mbases[m] + dl])
                accs[d % 4] = accs[d % 4] + hv * ev
            acc = (accs[0] + accs[1]) + (accs[2] + accs[3])
            score = acc * (1.0 / (MT + 1))
            out_v[pl.ds(off, L)] = 1.0 / (1.0 + jnp.exp(-score))

    issue(0, bufs[0])

    @pl.loop(0, NCHUNK, step=2)
    def _chunk(c):
        issue(c + 1, bufs[1])
        drain(bufs[0])
        compute(c, bufs[0])

        @pl.when(c + 2 < NCHUNK)
        def _():
            issue(c + 2, bufs[0])
        drain(bufs[1])
        compute(c + 1, bufs[1])

    pltpu.sync_copy(out_v, out_hbm.at[pl.ds(slot0, SPT)])


def kernel(h, target_index, negative_sample, item_emb, meta_emb,
           item_meta_indicies, item_meta_weights):
    idx_all = jnp.concatenate(
        [target_index[:, None], negative_sample], axis=1
    ).astype(jnp.int32).reshape(-1)
    hrow_map = ((jnp.arange(SPT, dtype=jnp.int32) // SLOTS) * DM).astype(jnp.int32)
    miw = jnp.concatenate(
        [
            item_meta_indicies.astype(jnp.float32),
            item_meta_weights,
            jnp.zeros((NUM_ITEMS, DM - 2 * MT), jnp.float32),
        ],
        axis=1,
    )

    scores = _score_kernel(
        h.reshape(-1), idx_all, item_emb, meta_emb.reshape(-1), miw, hrow_map,
    ).reshape(B, SLOTS)

    pos_out = scores[:, :1]
    neg_out = scores[:, 1:]
    pos_label = jnp.ones((B, 1), dtype=jnp.float32)
    neg_label = jnp.zeros((B, KNEG), dtype=jnp.float32)
    return pos_out, pos_label, neg_out, neg_label
